# Initial kernel scaffold; baseline (speedup 1.0000x reference)
import jax
import jax.numpy as jnp
from jax.experimental import pallas as pl


def kernel(x, edge_index, edge_weight, W1, b1, W2, b2, Wl1, bl1, Wl2, bl2, Wl3, bl3):
    raise NotImplementedError("write your pallas kernel here")


# speedup vs baseline: 1.7759x; 1.7759x over previous
import functools

import jax
import jax.numpy as jnp
from jax.experimental import pallas as pl
from jax.experimental.pallas import tpu as pltpu

N = 10000
RB = 2000


def _conv_mm_body(agg_ref, x_ref, dinv2_ref, W1_ref, b1_ref, W2_ref, out_ref):
    a = agg_ref[...] + dinv2_ref[...] * x_ref[...]
    h = jnp.maximum(
        jnp.dot(a, W1_ref[...], preferred_element_type=jnp.float32) + b1_ref[...],
        0.0,
    )
    out_ref[...] = jnp.dot(h, W2_ref[...], preferred_element_type=jnp.float32)


def _head_body(agg_ref, g_ref, dinv2_ref, b2_ref, Wl1_ref, bl1_ref,
               Wl2_ref, bl2_ref, Wl3_ref, bl3_ref, out_ref):
    a = agg_ref[...] + dinv2_ref[...] * g_ref[...] + b2_ref[...]
    h = jnp.maximum(a, 0.0)
    h = jnp.maximum(
        jnp.dot(h, Wl1_ref[...], preferred_element_type=jnp.float32) + bl1_ref[...], 0.0)
    h = jnp.maximum(
        jnp.dot(h, Wl2_ref[...], preferred_element_type=jnp.float32) + bl2_ref[...], 0.0)
    logits = jnp.dot(h, Wl3_ref[...], preferred_element_type=jnp.float32) + bl3_ref[...]
    m = jnp.max(logits, axis=1, keepdims=True)
    e = jnp.exp(logits - m)
    out_ref[...] = e / jnp.sum(e, axis=1, keepdims=True)


def _row_spec(cols):
    return pl.BlockSpec((RB, cols), lambda i: (i, 0))


def _full_spec(shape):
    return pl.BlockSpec(shape, lambda i: tuple(0 for _ in shape))


def _conv_mm(agg, x, dinv2, W1, b1, W2):
    return pl.pallas_call(
        _conv_mm_body,
        grid=(N // RB,),
        in_specs=[
            _row_spec(128), _row_spec(128), _row_spec(1),
            _full_spec((128, 512)), _full_spec((1, 512)), _full_spec((512, 256)),
        ],
        out_specs=_row_spec(256),
        out_shape=jax.ShapeDtypeStruct((N, 256), jnp.float32),
    )(agg, x, dinv2, W1, b1.reshape(1, 512), W2)


def _head(agg, g, dinv2, b2, Wl1, bl1, Wl2, bl2, Wl3, bl3):
    return pl.pallas_call(
        _head_body,
        grid=(N // RB,),
        in_specs=[
            _row_spec(256), _row_spec(256), _row_spec(1),
            _full_spec((1, 256)),
            _full_spec((256, 128)), _full_spec((1, 128)),
            _full_spec((128, 64)), _full_spec((1, 64)),
            _full_spec((64, 40)), _full_spec((1, 40)),
        ],
        out_specs=_row_spec(40),
        out_shape=jax.ShapeDtypeStruct((N, 40), jnp.float32),
    )(agg, g, dinv2, b2.reshape(1, 256), Wl1, bl1.reshape(1, 128),
      Wl2, bl2.reshape(1, 64), Wl3, bl3.reshape(1, 40))


def kernel(x, edge_index, edge_weight, W1, b1, W2, b2, Wl1, bl1, Wl2, bl2, Wl3, bl3):
    src = edge_index[0]
    dst = edge_index[1]
    deg = jax.ops.segment_sum(edge_weight, dst, num_segments=N) + 1.0
    dinv = jax.lax.rsqrt(deg)
    dinv2 = (dinv * dinv).reshape(N, 1)
    norm = dinv[src] * edge_weight * dinv[dst]

    agg1 = jax.ops.segment_sum(x[src] * norm[:, None], dst, num_segments=N)
    g = _conv_mm(agg1, x, dinv2, W1, b1, W2)
    agg2 = jax.ops.segment_sum(g[src] * norm[:, None], dst, num_segments=N)
    return _head(agg2, g, dinv2, b2, Wl1, bl1, Wl2, bl2, Wl3, bl3)


# speedup vs baseline: 3.5894x; 2.0211x over previous
import functools

import jax
import jax.numpy as jnp
from jax import lax
from jax.experimental import pallas as pl
from jax.experimental.pallas import tpu as pltpu
from jax.experimental.pallas import tpu_sc as plsc

N = 10000
E = 320000
NWORKERS = 32
EPT = E // NWORKERS
NV = N // 16
EV = EPT // 16
RB = 2000

_SC_MESH = plsc.VectorSubcoreMesh(core_axis_name="c", subcore_axis_name="s")
_SC_PARAMS = pltpu.CompilerParams(needs_layout_passes=False)


def _rsqrt16(d):
    bits = plsc.bitcast(d, jnp.int32)
    y = plsc.bitcast(jnp.int32(0x5F3759DF) - (bits >> 1), jnp.float32)
    for _ in range(4):
        y = y * (1.5 - 0.5 * d * y * y)
    return y


@functools.partial(
    pl.kernel,
    out_type=jax.ShapeDtypeStruct((NWORKERS, N), jnp.float32),
    mesh=_SC_MESH,
    compiler_params=_SC_PARAMS,
    scratch_types=[
        pltpu.VMEM((EPT,), jnp.int32),
        pltpu.VMEM((EPT,), jnp.float32),
        pltpu.VMEM((N,), jnp.float32),
    ],
)
def _deg_partials(dst_hbm, ew_hbm, out_hbm, dst_v, ew_v, deg_v):
    wid = lax.axis_index("s") * 2 + lax.axis_index("c")
    base = wid * EPT
    pltpu.sync_copy(dst_hbm.at[pl.ds(base, EPT)], dst_v)
    pltpu.sync_copy(ew_hbm.at[pl.ds(base, EPT)], ew_v)

    def zero_body(i, carry):
        deg_v[pl.ds(i * 16, 16)] = jnp.zeros((16,), jnp.float32)
        return carry

    lax.fori_loop(0, NV, zero_body, 0)

    def body(i, carry):
        idx = dst_v[pl.ds(i * 16, 16)]
        w = ew_v[pl.ds(i * 16, 16)]
        plsc.addupdate_scatter(deg_v, [idx], w)
        return carry

    lax.fori_loop(0, EV, body, 0)
    pltpu.sync_copy(deg_v, out_hbm.at[wid])


@functools.partial(
    pl.kernel,
    out_type=(
        jax.ShapeDtypeStruct((E,), jnp.float32),
        jax.ShapeDtypeStruct((N,), jnp.float32),
    ),
    mesh=_SC_MESH,
    compiler_params=_SC_PARAMS,
    scratch_types=[
        pltpu.VMEM((N,), jnp.float32),
        pltpu.VMEM((N,), jnp.float32),
        pltpu.VMEM((EPT,), jnp.int32),
        pltpu.VMEM((EPT,), jnp.int32),
        pltpu.VMEM((EPT,), jnp.float32),
        pltpu.VMEM((EPT,), jnp.float32),
    ],
)
def _edge_norm(parts_hbm, src_hbm, dst_hbm, ew_hbm, norm_hbm, dinv2_hbm,
               acc_v, tmp_v, src_v, dst_v, ew_v, norm_v):
    wid = lax.axis_index("s") * 2 + lax.axis_index("c")
    pltpu.sync_copy(parts_hbm.at[0], acc_v)

    def addp(p, carry):
        pltpu.sync_copy(parts_hbm.at[p], tmp_v)

        def add_body(i, c2):
            sl = pl.ds(i * 16, 16)
            acc_v[sl] = acc_v[sl] + tmp_v[sl]
            return c2

        return lax.fori_loop(0, NV, add_body, carry)

    lax.fori_loop(1, NWORKERS, addp, 0)

    def rsq_body(i, carry):
        sl = pl.ds(i * 16, 16)
        acc_v[sl] = _rsqrt16(acc_v[sl] + 1.0)
        return carry

    lax.fori_loop(0, NV, rsq_body, 0)

    @pl.when(wid == 0)
    def _():
        def d2_body(i, carry):
            sl = pl.ds(i * 16, 16)
            v = acc_v[sl]
            tmp_v[sl] = v * v
            return carry

        lax.fori_loop(0, NV, d2_body, 0)
        pltpu.sync_copy(tmp_v, dinv2_hbm)

    base = wid * EPT
    pltpu.sync_copy(src_hbm.at[pl.ds(base, EPT)], src_v)
    pltpu.sync_copy(dst_hbm.at[pl.ds(base, EPT)], dst_v)
    pltpu.sync_copy(ew_hbm.at[pl.ds(base, EPT)], ew_v)

    def nbody(i, carry):
        sl = pl.ds(i * 16, 16)
        a = plsc.load_gather(acc_v, [src_v[sl]])
        b = plsc.load_gather(acc_v, [dst_v[sl]])
        norm_v[sl] = a * ew_v[sl] * b
        return carry

    lax.fori_loop(0, EV, nbody, 0)
    pltpu.sync_copy(norm_v, norm_hbm.at[pl.ds(base, EPT)])


def _conv_mm_body(agg_ref, x_ref, dinv2_ref, W1_ref, b1_ref, W2_ref, out_ref):
    a = agg_ref[...] + dinv2_ref[...] * x_ref[...]
    h = jnp.maximum(
        jnp.dot(a, W1_ref[...], preferred_element_type=jnp.float32) + b1_ref[...],
        0.0,
    )
    out_ref[...] = jnp.dot(h, W2_ref[...], preferred_element_type=jnp.float32)


def _head_body(agg_ref, g_ref, dinv2_ref, b2_ref, Wl1_ref, bl1_ref,
               Wl2_ref, bl2_ref, Wl3_ref, bl3_ref, out_ref):
    a = agg_ref[...] + dinv2_ref[...] * g_ref[...] + b2_ref[...]
    h = jnp.maximum(a, 0.0)
    h = jnp.maximum(
        jnp.dot(h, Wl1_ref[...], preferred_element_type=jnp.float32) + bl1_ref[...], 0.0)
    h = jnp.maximum(
        jnp.dot(h, Wl2_ref[...], preferred_element_type=jnp.float32) + bl2_ref[...], 0.0)
    logits = jnp.dot(h, Wl3_ref[...], preferred_element_type=jnp.float32) + bl3_ref[...]
    m = jnp.max(logits, axis=1, keepdims=True)
    e = jnp.exp(logits - m)
    out_ref[...] = e / jnp.sum(e, axis=1, keepdims=True)


def _row_spec(cols):
    return pl.BlockSpec((RB, cols), lambda i: (i, 0))


def _full_spec(shape):
    return pl.BlockSpec(shape, lambda i: tuple(0 for _ in shape))


def _conv_mm(agg, x, dinv2, W1, b1, W2):
    return pl.pallas_call(
        _conv_mm_body,
        grid=(N // RB,),
        in_specs=[
            _row_spec(128), _row_spec(128), _row_spec(1),
            _full_spec((128, 512)), _full_spec((1, 512)), _full_spec((512, 256)),
        ],
        out_specs=_row_spec(256),
        out_shape=jax.ShapeDtypeStruct((N, 256), jnp.float32),
    )(agg, x, dinv2, W1, b1.reshape(1, 512), W2)


def _head(agg, g, dinv2, b2, Wl1, bl1, Wl2, bl2, Wl3, bl3):
    return pl.pallas_call(
        _head_body,
        grid=(N // RB,),
        in_specs=[
            _row_spec(256), _row_spec(256), _row_spec(1),
            _full_spec((1, 256)),
            _full_spec((256, 128)), _full_spec((1, 128)),
            _full_spec((128, 64)), _full_spec((1, 64)),
            _full_spec((64, 40)), _full_spec((1, 40)),
        ],
        out_specs=_row_spec(40),
        out_shape=jax.ShapeDtypeStruct((N, 40), jnp.float32),
    )(agg, g, dinv2, b2.reshape(1, 256), Wl1, bl1.reshape(1, 128),
      Wl2, bl2.reshape(1, 64), Wl3, bl3.reshape(1, 40))


def kernel(x, edge_index, edge_weight, W1, b1, W2, b2, Wl1, bl1, Wl2, bl2, Wl3, bl3):
    src = edge_index[0]
    dst = edge_index[1]
    parts = _deg_partials(dst, edge_weight)
    norm, dinv2 = _edge_norm(parts, src, dst, edge_weight)
    dinv2 = dinv2.reshape(N, 1)

    agg1 = jax.ops.segment_sum(x[src] * norm[:, None], dst, num_segments=N)
    g = _conv_mm(agg1, x, dinv2, W1, b1, W2)
    agg2 = jax.ops.segment_sum(g[src] * norm[:, None], dst, num_segments=N)
    return _head(agg2, g, dinv2, b2, Wl1, bl1, Wl2, bl2, Wl3, bl3)


# speedup vs baseline: 19.2660x; 5.3674x over previous
import functools

import jax
import jax.numpy as jnp
from jax import lax
from jax.experimental import pallas as pl
from jax.experimental.pallas import tpu as pltpu
from jax.experimental.pallas import tpu_sc as plsc

N = 10000
E = 320000
NWORKERS = 32
EPT = E // NWORKERS
NV = N // 16
EV = EPT // 16
RB = 2000

_SC_MESH = plsc.VectorSubcoreMesh(core_axis_name="c", subcore_axis_name="s")
_SC_PARAMS = pltpu.CompilerParams(needs_layout_passes=False)


def _rsqrt16(d):
    bits = plsc.bitcast(d, jnp.int32)
    y = plsc.bitcast(jnp.int32(0x5F3759DF) - (bits >> 1), jnp.float32)
    for _ in range(4):
        y = y * (1.5 - 0.5 * d * y * y)
    return y


@functools.partial(
    pl.kernel,
    out_type=jax.ShapeDtypeStruct((NWORKERS, N), jnp.float32),
    mesh=_SC_MESH,
    compiler_params=_SC_PARAMS,
    scratch_types=[
        pltpu.VMEM((EPT,), jnp.int32),
        pltpu.VMEM((EPT,), jnp.float32),
        pltpu.VMEM((N,), jnp.float32),
    ],
)
def _deg_partials(dst_hbm, ew_hbm, out_hbm, dst_v, ew_v, deg_v):
    wid = lax.axis_index("s") * 2 + lax.axis_index("c")
    base = wid * EPT
    pltpu.sync_copy(dst_hbm.at[pl.ds(base, EPT)], dst_v)
    pltpu.sync_copy(ew_hbm.at[pl.ds(base, EPT)], ew_v)

    def zero_body(i, carry):
        deg_v[pl.ds(i * 16, 16)] = jnp.zeros((16,), jnp.float32)
        return carry

    lax.fori_loop(0, NV, zero_body, 0)

    def body(i, carry):
        idx = dst_v[pl.ds(i * 16, 16)]
        w = ew_v[pl.ds(i * 16, 16)]
        plsc.addupdate_scatter(deg_v, [idx], w)
        return carry

    lax.fori_loop(0, EV, body, 0)
    pltpu.sync_copy(deg_v, out_hbm.at[wid])


@functools.partial(
    pl.kernel,
    out_type=(
        jax.ShapeDtypeStruct((E,), jnp.float32),
        jax.ShapeDtypeStruct((N,), jnp.float32),
    ),
    mesh=_SC_MESH,
    compiler_params=_SC_PARAMS,
    scratch_types=[
        pltpu.VMEM((N,), jnp.float32),
        pltpu.VMEM((N,), jnp.float32),
        pltpu.VMEM((EPT,), jnp.int32),
        pltpu.VMEM((EPT,), jnp.int32),
        pltpu.VMEM((EPT,), jnp.float32),
        pltpu.VMEM((EPT,), jnp.float32),
    ],
)
def _edge_norm(parts_hbm, src_hbm, dst_hbm, ew_hbm, norm_hbm, dinv2_hbm,
               acc_v, tmp_v, src_v, dst_v, ew_v, norm_v):
    wid = lax.axis_index("s") * 2 + lax.axis_index("c")
    pltpu.sync_copy(parts_hbm.at[0], acc_v)

    def addp(p, carry):
        pltpu.sync_copy(parts_hbm.at[p], tmp_v)

        def add_body(i, c2):
            sl = pl.ds(i * 16, 16)
            acc_v[sl] = acc_v[sl] + tmp_v[sl]
            return c2

        return lax.fori_loop(0, NV, add_body, carry)

    lax.fori_loop(1, NWORKERS, addp, 0)

    def rsq_body(i, carry):
        sl = pl.ds(i * 16, 16)
        acc_v[sl] = _rsqrt16(acc_v[sl] + 1.0)
        return carry

    lax.fori_loop(0, NV, rsq_body, 0)

    @pl.when(wid == 0)
    def _():
        def d2_body(i, carry):
            sl = pl.ds(i * 16, 16)
            v = acc_v[sl]
            tmp_v[sl] = v * v
            return carry

        lax.fori_loop(0, NV, d2_body, 0)
        pltpu.sync_copy(tmp_v, dinv2_hbm)

    base = wid * EPT
    pltpu.sync_copy(src_hbm.at[pl.ds(base, EPT)], src_v)
    pltpu.sync_copy(dst_hbm.at[pl.ds(base, EPT)], dst_v)
    pltpu.sync_copy(ew_hbm.at[pl.ds(base, EPT)], ew_v)

    def nbody(i, carry):
        sl = pl.ds(i * 16, 16)
        a = plsc.load_gather(acc_v, [src_v[sl]])
        b = plsc.load_gather(acc_v, [dst_v[sl]])
        norm_v[sl] = a * ew_v[sl] * b
        return carry

    lax.fori_loop(0, EV, nbody, 0)
    pltpu.sync_copy(norm_v, norm_hbm.at[pl.ds(base, EPT)])


BK = 80
GB = 25


def _make_aggregate(split_edges):
    D2 = 128
    nsplit = 32 if split_edges else 16
    ept = E // nsplit
    nb = ept // BK
    ngroups = nb // GB

    @functools.partial(
        pl.kernel,
        out_type=jax.ShapeDtypeStruct((2, N, D2), jnp.float32),
        mesh=_SC_MESH,
        compiler_params=_SC_PARAMS,
        scratch_types=[
            pltpu.VMEM((GB * BK,), jnp.int32),
            pltpu.VMEM((GB, BK), jnp.int32),
            pltpu.VMEM((GB * BK,), jnp.float32),
            pltpu.VMEM((2, BK, D2), jnp.float32),
            pltpu.VMEM_SHARED((N, D2), jnp.float32),
            pltpu.SemaphoreType.DMA,
            pltpu.SemaphoreType.DMA,
        ],
    )
    def agg(table_hbm, src_hbm, dst4_hbm, norm_hbm, out_hbm,
            src_v, dst_v, norm_v, rows_v, shared, sg0, sg1):
        c = lax.axis_index("c")
        s = lax.axis_index("s")
        eslot = s * 2 + c if split_edges else s
        ebase = eslot * ept

        def zrow(j, carry):
            for k in range(D2 // 16):
                rows_v[0, j, pl.ds(k * 16, 16)] = jnp.zeros((16,), jnp.float32)
            return carry

        lax.fori_loop(0, BK, zrow, 0)
        nbase = s * 640
        nchunks = jnp.where(s < 15, 8, 5)

        def zchunk(j, carry):
            pltpu.sync_copy(rows_v.at[0], shared.at[pl.ds(nbase + j * BK, BK)])
            return carry

        lax.fori_loop(0, nchunks, zchunk, 0)
        plsc.subcore_barrier()

        tbl = table_hbm if split_edges else table_hbm.at[c]
        sgs = (sg0, sg1)

        def gather(i, b):
            return pltpu.async_copy(
                tbl.at[src_v.at[pl.ds(i * BK, BK)]], rows_v.at[b], sgs[b])

        def consume(i, b):
            pltpu.make_async_copy(
                tbl.at[src_v.at[pl.ds(i * BK, BK)]], rows_v.at[b], sgs[b]).wait()

            def ebody(e, carry):
                nsp = plsc.load_gather(
                    norm_v, [jnp.full((16,), i * BK + e, jnp.int32)])
                for k in range(D2 // 16):
                    sl = pl.ds(k * 16, 16)
                    rows_v[b, e, sl] = rows_v[b, e, sl] * nsp
                return carry

            lax.fori_loop(0, BK, ebody, 0, unroll=8)
            pltpu.sync_copy(rows_v.at[b], shared.at[dst_v.at[i]], add=True)

        def group_body(g, carry):
            gb = pl.multiple_of(ebase + g * (GB * BK), 8)
            pltpu.sync_copy(src_hbm.at[pl.ds(gb, GB * BK)], src_v)
            pltpu.sync_copy(dst4_hbm.at[eslot, g], dst_v)
            pltpu.sync_copy(norm_hbm.at[pl.ds(gb, GB * BK)], norm_v)
            gather(0, 0)
            gather(1, 1)

            def pair_body(blk, carry2):
                for b in (0, 1):
                    i = blk * 2 + b
                    consume(i, b)

                    @pl.when(i + 2 < GB)
                    def _():
                        gather(i + 2, b)
                return carry2

            lax.fori_loop(0, GB // 2, pair_body, 0)
            consume(GB - 1, (GB - 1) % 2)
            return carry

        lax.fori_loop(0, ngroups, group_body, 0)
        plsc.subcore_barrier()
        out_c = out_hbm.at[c]

        def wchunk(j, carry):
            sl = pl.ds(nbase + j * BK, BK)
            pltpu.sync_copy(shared.at[sl], out_c.at[sl])
            return carry

        lax.fori_loop(0, nchunks, wchunk, 0)

    return agg


_agg_conv1 = _make_aggregate(True)
_agg_conv2 = _make_aggregate(False)


def _conv_mm_body(agg_ref, x_ref, dinv2_ref, W1_ref, b1_ref, W2_ref, out_ref):
    a = agg_ref[0] + agg_ref[1] + dinv2_ref[...] * x_ref[...]
    h = jnp.maximum(
        jnp.dot(a, W1_ref[...], preferred_element_type=jnp.float32) + b1_ref[...],
        0.0,
    )
    g = jnp.dot(h, W2_ref[...], preferred_element_type=jnp.float32)
    out_ref[0] = g[:, :128]
    out_ref[1] = g[:, 128:]


def _head_body(agg_ref, g_ref, dinv2_ref, b2_ref, Wl1_ref, bl1_ref,
               Wl2_ref, bl2_ref, Wl3_ref, bl3_ref, out_ref):
    d2 = dinv2_ref[...]
    h0 = jnp.maximum(agg_ref[0] + d2 * g_ref[0] + b2_ref[:, :128], 0.0)
    h1 = jnp.maximum(agg_ref[1] + d2 * g_ref[1] + b2_ref[:, 128:], 0.0)
    h = (jnp.dot(h0, Wl1_ref[:128], preferred_element_type=jnp.float32)
         + jnp.dot(h1, Wl1_ref[128:], preferred_element_type=jnp.float32)
         + bl1_ref[...])
    h = jnp.maximum(h, 0.0)
    h = jnp.maximum(
        jnp.dot(h, Wl2_ref[...], preferred_element_type=jnp.float32) + bl2_ref[...], 0.0)
    logits = jnp.dot(h, Wl3_ref[...], preferred_element_type=jnp.float32) + bl3_ref[...]
    m = jnp.max(logits, axis=1, keepdims=True)
    e = jnp.exp(logits - m)
    out_ref[...] = e / jnp.sum(e, axis=1, keepdims=True)


def _row_spec(cols):
    return pl.BlockSpec((RB, cols), lambda i: (i, 0))


def _full_spec(shape):
    return pl.BlockSpec(shape, lambda i: tuple(0 for _ in shape))


def _split_spec(cols):
    return pl.BlockSpec((2, RB, cols), lambda i: (0, i, 0))


def _conv_mm(agg2, x, dinv2, W1, b1, W2):
    return pl.pallas_call(
        _conv_mm_body,
        grid=(N // RB,),
        in_specs=[
            _split_spec(128), _row_spec(128), _row_spec(1),
            _full_spec((128, 512)), _full_spec((1, 512)), _full_spec((512, 256)),
        ],
        out_specs=_split_spec(128),
        out_shape=jax.ShapeDtypeStruct((2, N, 128), jnp.float32),
    )(agg2, x, dinv2, W1, b1.reshape(1, 512), W2)


def _head(agg, g, dinv2, b2, Wl1, bl1, Wl2, bl2, Wl3, bl3):
    return pl.pallas_call(
        _head_body,
        grid=(N // RB,),
        in_specs=[
            _split_spec(128), _split_spec(128), _row_spec(1),
            _full_spec((1, 256)),
            _full_spec((256, 128)), _full_spec((1, 128)),
            _full_spec((128, 64)), _full_spec((1, 64)),
            _full_spec((64, 40)), _full_spec((1, 40)),
        ],
        out_specs=_row_spec(40),
        out_shape=jax.ShapeDtypeStruct((N, 40), jnp.float32),
    )(agg, g, dinv2, b2.reshape(1, 256), Wl1, bl1.reshape(1, 128),
      Wl2, bl2.reshape(1, 64), Wl3, bl3.reshape(1, 40))


def kernel(x, edge_index, edge_weight, W1, b1, W2, b2, Wl1, bl1, Wl2, bl2, Wl3, bl3):
    src = edge_index[0]
    dst = edge_index[1]
    parts = _deg_partials(dst, edge_weight)
    norm, dinv2 = _edge_norm(parts, src, dst, edge_weight)
    dinv2 = dinv2.reshape(N, 1)

    agg1 = _agg_conv1(x, src, dst.reshape(32, E // (32 * GB * BK), GB, BK), norm)
    g2 = _conv_mm(agg1, x, dinv2, W1, b1, W2)
    agg2 = _agg_conv2(g2, src, dst.reshape(16, E // (16 * GB * BK), GB, BK), norm)
    return _head(agg2, g2, dinv2, b2, Wl1, bl1, Wl2, bl2, Wl3, bl3)


# speedup vs baseline: 21.4716x; 1.1145x over previous
import functools

import jax
import jax.numpy as jnp
from jax import lax
from jax.experimental import pallas as pl
from jax.experimental.pallas import tpu as pltpu
from jax.experimental.pallas import tpu_sc as plsc

N = 10000
NP = 10240
E = 320000
NWORKERS = 32
EPT = E // NWORKERS
NV = N // 16
EV = EPT // 16
RB = 2000

_SC_MESH = plsc.VectorSubcoreMesh(core_axis_name="c", subcore_axis_name="s")
_SC_PARAMS = pltpu.CompilerParams(needs_layout_passes=False)


def _rsqrt16(d):
    bits = plsc.bitcast(d, jnp.int32)
    y = plsc.bitcast(jnp.int32(0x5F3759DF) - (bits >> 1), jnp.float32)
    for _ in range(4):
        y = y * (1.5 - 0.5 * d * y * y)
    return y


@functools.partial(
    pl.kernel,
    out_type=jax.ShapeDtypeStruct((NWORKERS, NP), jnp.float32),
    mesh=_SC_MESH,
    compiler_params=_SC_PARAMS,
    scratch_types=[
        pltpu.VMEM((EPT,), jnp.int32),
        pltpu.VMEM((EPT,), jnp.float32),
        pltpu.VMEM((NP,), jnp.float32),
    ],
)
def _deg_partials(dst_hbm, ew_hbm, out_hbm, dst_v, ew_v, deg_v):
    wid = lax.axis_index("s") * 2 + lax.axis_index("c")
    base = wid * EPT
    pltpu.sync_copy(dst_hbm.at[pl.ds(base, EPT)], dst_v)
    pltpu.sync_copy(ew_hbm.at[pl.ds(base, EPT)], ew_v)

    def zero_body(i, carry):
        deg_v[pl.ds(i * 16, 16)] = jnp.zeros((16,), jnp.float32)
        return carry

    lax.fori_loop(0, NP // 16, zero_body, 0, unroll=8)

    def body(i, carry):
        idx = dst_v[pl.ds(i * 16, 16)]
        w = ew_v[pl.ds(i * 16, 16)]
        plsc.addupdate_scatter(deg_v, [idx], w)
        return carry

    lax.fori_loop(0, EV, body, 0, unroll=4)
    pltpu.sync_copy(deg_v, out_hbm.at[wid])


@functools.partial(
    pl.kernel,
    out_type=(
        jax.ShapeDtypeStruct((E,), jnp.float32),
        jax.ShapeDtypeStruct((N,), jnp.float32),
    ),
    mesh=_SC_MESH,
    compiler_params=_SC_PARAMS,
    scratch_types=[
        pltpu.VMEM((NP,), jnp.float32),
        pltpu.VMEM((640,), jnp.float32),
        pltpu.VMEM((640,), jnp.float32),
        pltpu.VMEM((EPT,), jnp.int32),
        pltpu.VMEM((EPT,), jnp.int32),
        pltpu.VMEM((EPT,), jnp.float32),
        pltpu.VMEM((EPT,), jnp.float32),
        pltpu.VMEM_SHARED((NP,), jnp.float32),
    ],
)
def _edge_norm(parts_hbm, src_hbm, dst_hbm, ew_hbm, norm_hbm, dinv_hbm,
               acc_v, st_v, tmp_v, src_v, dst_v, ew_v, norm_v, dinv_sh):
    c = lax.axis_index("c")
    s = lax.axis_index("s")
    wid = s * 2 + c
    nbase = s * 640

    def addp(p, carry):
        pltpu.sync_copy(parts_hbm.at[p].at[pl.ds(nbase, 640)], tmp_v)

        def add_body(i, c2):
            sl = pl.ds(i * 16, 16)
            st_v[sl] = st_v[sl] + tmp_v[sl]
            return c2

        return lax.fori_loop(0, 40, add_body, carry, unroll=8)

    pltpu.sync_copy(parts_hbm.at[0].at[pl.ds(nbase, 640)], st_v)
    lax.fori_loop(1, NWORKERS, addp, 0)

    def rsq_body(i, carry):
        sl = pl.ds(i * 16, 16)
        st_v[sl] = _rsqrt16(st_v[sl] + 1.0)
        return carry

    lax.fori_loop(0, 40, rsq_body, 0, unroll=4)
    pltpu.sync_copy(st_v, dinv_sh.at[pl.ds(nbase, 640)])
    plsc.subcore_barrier()
    pltpu.sync_copy(dinv_sh, acc_v)

    @pl.when(wid == 0)
    def _():
        pltpu.sync_copy(acc_v.at[pl.ds(0, N)], dinv_hbm)

    base = wid * EPT
    pltpu.sync_copy(src_hbm.at[pl.ds(base, EPT)], src_v)
    pltpu.sync_copy(dst_hbm.at[pl.ds(base, EPT)], dst_v)
    pltpu.sync_copy(ew_hbm.at[pl.ds(base, EPT)], ew_v)

    def nbody(i, carry):
        sl = pl.ds(i * 16, 16)
        a = plsc.load_gather(acc_v, [src_v[sl]])
        b = plsc.load_gather(acc_v, [dst_v[sl]])
        norm_v[sl] = a * ew_v[sl] * b
        return carry

    lax.fori_loop(0, EV, nbody, 0, unroll=8)
    pltpu.sync_copy(norm_v, norm_hbm.at[pl.ds(base, EPT)])


BK = 80
GB = 25


def _make_aggregate(split_edges):
    D2 = 128
    nsplit = 32 if split_edges else 16
    ept = E // nsplit
    nb = ept // BK
    ngroups = nb // GB

    @functools.partial(
        pl.kernel,
        out_type=jax.ShapeDtypeStruct((2, N, D2), jnp.float32),
        mesh=_SC_MESH,
        compiler_params=_SC_PARAMS,
        scratch_types=[
            pltpu.VMEM((GB * BK,), jnp.int32),
            pltpu.VMEM((GB, BK), jnp.int32),
            pltpu.VMEM((GB * BK,), jnp.float32),
            pltpu.VMEM((3, BK, D2), jnp.float32),
            pltpu.VMEM_SHARED((N, D2), jnp.float32),
            pltpu.SemaphoreType.DMA,
            pltpu.SemaphoreType.DMA,
            pltpu.SemaphoreType.DMA,
            pltpu.SemaphoreType.DMA,
            pltpu.SemaphoreType.DMA,
            pltpu.SemaphoreType.DMA,
        ],
    )
    def agg(table_hbm, src_hbm, dst4_hbm, norm_hbm, out_hbm,
            src_v, dst_v, norm_v, rows_v, shared, sg0, sg1, sg2, ss0, ss1, ss2):
        c = lax.axis_index("c")
        s = lax.axis_index("s")
        eslot = s * 2 + c if split_edges else s
        ebase = eslot * ept

        def zrow(j, carry):
            for k in range(D2 // 16):
                rows_v[0, j, pl.ds(k * 16, 16)] = jnp.zeros((16,), jnp.float32)
            return carry

        lax.fori_loop(0, BK, zrow, 0)
        nbase = s * 640
        nchunks = jnp.where(s < 15, 8, 5)

        def zchunk(j, carry):
            pltpu.sync_copy(rows_v.at[0], shared.at[pl.ds(nbase + j * BK, BK)])
            return carry

        lax.fori_loop(0, nchunks, zchunk, 0)
        plsc.subcore_barrier()

        tbl = table_hbm if split_edges else table_hbm.at[c]
        sgs = (sg0, sg1, sg2)
        sss = (ss0, ss1, ss2)

        def gather(i, b):
            return pltpu.async_copy(
                tbl.at[src_v.at[pl.ds(i * BK, BK)]], rows_v.at[b], sgs[b])

        def wait_gather(i, b):
            pltpu.make_async_copy(
                tbl.at[src_v.at[pl.ds(i * BK, BK)]], rows_v.at[b], sgs[b]).wait()

        def wait_scatter(i, b):
            pltpu.make_async_copy(
                rows_v.at[b], shared.at[dst_v.at[i]], sss[b]).wait()

        def scale(i, b):
            def ebody(e, carry):
                nsp = plsc.load_gather(
                    norm_v, [jnp.full((16,), i * BK + e, jnp.int32)])
                for k in range(D2 // 16):
                    sl = pl.ds(k * 16, 16)
                    rows_v[b, e, sl] = rows_v[b, e, sl] * nsp
                return carry

            lax.fori_loop(0, BK, ebody, 0, unroll=8)

        def group_body(g, carry):
            @pl.when(g > 0)
            def _():
                wait_scatter(GB - 3, 1)
                wait_scatter(GB - 2, 2)

            gb = pl.multiple_of(ebase + g * (GB * BK), 8)
            pltpu.sync_copy(src_hbm.at[pl.ds(gb, GB * BK)], src_v)
            pltpu.sync_copy(dst4_hbm.at[eslot, g], dst_v)
            pltpu.sync_copy(norm_hbm.at[pl.ds(gb, GB * BK)], norm_v)
            gather(0, 0)
            gather(1, 1)
            gather(2, 2)

            def tri_body(blk, carry2):
                base_i = blk * 3
                for b in (0, 1, 2):
                    i = base_i + b
                    wait_gather(i, b)
                    scale(i, b)
                    pltpu.async_copy(rows_v.at[b], shared.at[dst_v.at[i]],
                                     sss[b], add=True)
                for b in (0, 1, 2):
                    i = base_i + b

                    @pl.when(i + 3 < GB)
                    def _():
                        wait_scatter(i, b)
                        gather(i + 3, b)
                return carry2

            lax.fori_loop(0, GB // 3, tri_body, 0)
            wait_gather(GB - 1, 0)
            scale(GB - 1, 0)
            pltpu.sync_copy(rows_v.at[0], shared.at[dst_v.at[GB - 1]], add=True)
            return carry

        lax.fori_loop(0, ngroups, group_body, 0)
        wait_scatter(GB - 3, 1)
        wait_scatter(GB - 2, 2)
        plsc.subcore_barrier()
        out_c = out_hbm.at[c]

        def wchunk(j, carry):
            sl = pl.ds(nbase + j * BK, BK)
            pltpu.sync_copy(shared.at[sl], out_c.at[sl])
            return carry

        lax.fori_loop(0, nchunks, wchunk, 0)

    return agg


_agg_conv1 = _make_aggregate(True)
_agg_conv2 = _make_aggregate(False)


def _conv_mm_body(agg_ref, x_ref, dinv_ref, W1_ref, b1_ref, W2_ref, out_ref):
    d = dinv_ref[...]
    a = agg_ref[0] + agg_ref[1] + (d * d) * x_ref[...]
    h = jnp.maximum(
        jnp.dot(a, W1_ref[...], preferred_element_type=jnp.float32) + b1_ref[...],
        0.0,
    )
    g = jnp.dot(h, W2_ref[...], preferred_element_type=jnp.float32)
    out_ref[0] = g[:, :128]
    out_ref[1] = g[:, 128:]


def _head_body(agg_ref, g_ref, dinv_ref, b2_ref, Wl1_ref, bl1_ref,
               Wl2_ref, bl2_ref, Wl3_ref, bl3_ref, out_ref):
    d = dinv_ref[...]
    d2 = d * d
    h0 = jnp.maximum(agg_ref[0] + d2 * g_ref[0] + b2_ref[:, :128], 0.0)
    h1 = jnp.maximum(agg_ref[1] + d2 * g_ref[1] + b2_ref[:, 128:], 0.0)
    h = (jnp.dot(h0, Wl1_ref[:128], preferred_element_type=jnp.float32)
         + jnp.dot(h1, Wl1_ref[128:], preferred_element_type=jnp.float32)
         + bl1_ref[...])
    h = jnp.maximum(h, 0.0)
    h = jnp.maximum(
        jnp.dot(h, Wl2_ref[...], preferred_element_type=jnp.float32) + bl2_ref[...], 0.0)
    logits = jnp.dot(h, Wl3_ref[...], preferred_element_type=jnp.float32) + bl3_ref[...]
    m = jnp.max(logits, axis=1, keepdims=True)
    e = jnp.exp(logits - m)
    out_ref[...] = e / jnp.sum(e, axis=1, keepdims=True)


def _row_spec(cols):
    return pl.BlockSpec((RB, cols), lambda i: (i, 0))


def _full_spec(shape):
    return pl.BlockSpec(shape, lambda i: tuple(0 for _ in shape))


def _split_spec(cols):
    return pl.BlockSpec((2, RB, cols), lambda i: (0, i, 0))


def _conv_mm(agg2, x, dinv2, W1, b1, W2):
    return pl.pallas_call(
        _conv_mm_body,
        grid=(N // RB,),
        in_specs=[
            _split_spec(128), _row_spec(128), _row_spec(1),
            _full_spec((128, 512)), _full_spec((1, 512)), _full_spec((512, 256)),
        ],
        out_specs=_split_spec(128),
        out_shape=jax.ShapeDtypeStruct((2, N, 128), jnp.float32),
    )(agg2, x, dinv2, W1, b1.reshape(1, 512), W2)


def _head(agg, g, dinv2, b2, Wl1, bl1, Wl2, bl2, Wl3, bl3):
    return pl.pallas_call(
        _head_body,
        grid=(N // RB,),
        in_specs=[
            _split_spec(128), _split_spec(128), _row_spec(1),
            _full_spec((1, 256)),
            _full_spec((256, 128)), _full_spec((1, 128)),
            _full_spec((128, 64)), _full_spec((1, 64)),
            _full_spec((64, 40)), _full_spec((1, 40)),
        ],
        out_specs=_row_spec(40),
        out_shape=jax.ShapeDtypeStruct((N, 40), jnp.float32),
    )(agg, g, dinv2, b2.reshape(1, 256), Wl1, bl1.reshape(1, 128),
      Wl2, bl2.reshape(1, 64), Wl3, bl3.reshape(1, 40))


def kernel(x, edge_index, edge_weight, W1, b1, W2, b2, Wl1, bl1, Wl2, bl2, Wl3, bl3):
    src = edge_index[0]
    dst = edge_index[1]
    parts = _deg_partials(dst, edge_weight)
    norm, dinv = _edge_norm(parts, src, dst, edge_weight)
    dinv = dinv.reshape(N, 1)

    agg1 = _agg_conv1(x, src, dst.reshape(32, E // (32 * GB * BK), GB, BK), norm)
    g2 = _conv_mm(agg1, x, dinv, W1, b1, W2)
    agg2 = _agg_conv2(g2, src, dst.reshape(16, E // (16 * GB * BK), GB, BK), norm)
    return _head(agg2, g2, dinv, b2, Wl1, bl1, Wl2, bl2, Wl3, bl3)


# speedup vs baseline: 22.7596x; 1.0600x over previous
import functools

import jax
import jax.numpy as jnp
from jax import lax
from jax.experimental import pallas as pl
from jax.experimental.pallas import tpu as pltpu
from jax.experimental.pallas import tpu_sc as plsc

N = 10000
NP = 10240
E = 320000
NWORKERS = 32
EPT = E // NWORKERS
NV = N // 16
EV = EPT // 16
RB = 2000

_SC_MESH = plsc.VectorSubcoreMesh(core_axis_name="c", subcore_axis_name="s")
_SC_PARAMS = pltpu.CompilerParams(needs_layout_passes=False)


def _rsqrt16(d):
    bits = plsc.bitcast(d, jnp.int32)
    y = plsc.bitcast(jnp.int32(0x5F3759DF) - (bits >> 1), jnp.float32)
    for _ in range(4):
        y = y * (1.5 - 0.5 * d * y * y)
    return y


@functools.partial(
    pl.kernel,
    out_type=jax.ShapeDtypeStruct((NWORKERS, NP), jnp.float32),
    mesh=_SC_MESH,
    compiler_params=_SC_PARAMS,
    scratch_types=[
        pltpu.VMEM((EPT,), jnp.int32),
        pltpu.VMEM((EPT,), jnp.float32),
        pltpu.VMEM((NP,), jnp.float32),
    ],
)
def _deg_partials(dst_hbm, ew_hbm, out_hbm, dst_v, ew_v, deg_v):
    wid = lax.axis_index("s") * 2 + lax.axis_index("c")
    base = wid * EPT
    pltpu.sync_copy(dst_hbm.at[pl.ds(base, EPT)], dst_v)
    pltpu.sync_copy(ew_hbm.at[pl.ds(base, EPT)], ew_v)

    def zero_body(i, carry):
        deg_v[pl.ds(i * 16, 16)] = jnp.zeros((16,), jnp.float32)
        return carry

    lax.fori_loop(0, NP // 16, zero_body, 0, unroll=8)

    def body(i, carry):
        idx = dst_v[pl.ds(i * 16, 16)]
        w = ew_v[pl.ds(i * 16, 16)]
        plsc.addupdate_scatter(deg_v, [idx], w)
        return carry

    lax.fori_loop(0, EV, body, 0, unroll=4)
    pltpu.sync_copy(deg_v, out_hbm.at[wid])


@functools.partial(
    pl.kernel,
    out_type=(
        jax.ShapeDtypeStruct((E,), jnp.float32),
        jax.ShapeDtypeStruct((N,), jnp.float32),
    ),
    mesh=_SC_MESH,
    compiler_params=_SC_PARAMS,
    scratch_types=[
        pltpu.VMEM((NP,), jnp.float32),
        pltpu.VMEM((640,), jnp.float32),
        pltpu.VMEM((640,), jnp.float32),
        pltpu.VMEM((EPT,), jnp.int32),
        pltpu.VMEM((EPT,), jnp.int32),
        pltpu.VMEM((EPT,), jnp.float32),
        pltpu.VMEM((EPT,), jnp.float32),
        pltpu.VMEM_SHARED((NP,), jnp.float32),
    ],
)
def _edge_norm(parts_hbm, src_hbm, dst_hbm, ew_hbm, norm_hbm, dinv_hbm,
               acc_v, st_v, tmp_v, src_v, dst_v, ew_v, norm_v, dinv_sh):
    c = lax.axis_index("c")
    s = lax.axis_index("s")
    wid = s * 2 + c
    nbase = s * 640

    def addp(p, carry):
        pltpu.sync_copy(parts_hbm.at[p].at[pl.ds(nbase, 640)], tmp_v)

        def add_body(i, c2):
            sl = pl.ds(i * 16, 16)
            st_v[sl] = st_v[sl] + tmp_v[sl]
            return c2

        return lax.fori_loop(0, 40, add_body, carry, unroll=8)

    pltpu.sync_copy(parts_hbm.at[0].at[pl.ds(nbase, 640)], st_v)
    lax.fori_loop(1, NWORKERS, addp, 0)

    def rsq_body(i, carry):
        sl = pl.ds(i * 16, 16)
        st_v[sl] = _rsqrt16(st_v[sl] + 1.0)
        return carry

    lax.fori_loop(0, 40, rsq_body, 0, unroll=4)
    pltpu.sync_copy(st_v, dinv_sh.at[pl.ds(nbase, 640)])
    plsc.subcore_barrier()
    pltpu.sync_copy(dinv_sh, acc_v)

    @pl.when(wid == 0)
    def _():
        pltpu.sync_copy(acc_v.at[pl.ds(0, N)], dinv_hbm)

    base = wid * EPT
    pltpu.sync_copy(src_hbm.at[pl.ds(base, EPT)], src_v)
    pltpu.sync_copy(dst_hbm.at[pl.ds(base, EPT)], dst_v)
    pltpu.sync_copy(ew_hbm.at[pl.ds(base, EPT)], ew_v)

    def nbody(i, carry):
        sl = pl.ds(i * 16, 16)
        a = plsc.load_gather(acc_v, [src_v[sl]])
        b = plsc.load_gather(acc_v, [dst_v[sl]])
        norm_v[sl] = a * ew_v[sl] * b
        return carry

    lax.fori_loop(0, EV, nbody, 0, unroll=8)
    pltpu.sync_copy(norm_v, norm_hbm.at[pl.ds(base, EPT)])


BK = 80
GB = 25


def _make_aggregate(split_edges):
    D2 = 128
    nsplit = 32 if split_edges else 16
    ept = E // nsplit
    nb = ept // BK
    ngroups = nb // GB

    @functools.partial(
        pl.kernel,
        out_type=jax.ShapeDtypeStruct((2, N, D2), jnp.float32),
        mesh=_SC_MESH,
        compiler_params=_SC_PARAMS,
        scratch_types=[
            pltpu.VMEM((GB * BK,), jnp.int32),
            pltpu.VMEM((GB, BK), jnp.int32),
            pltpu.VMEM((GB * BK,), jnp.float32),
            pltpu.VMEM((3, BK, D2), jnp.float32),
            pltpu.VMEM_SHARED((N, D2), jnp.float32),
            pltpu.SemaphoreType.DMA,
            pltpu.SemaphoreType.DMA,
            pltpu.SemaphoreType.DMA,
            pltpu.SemaphoreType.DMA,
            pltpu.SemaphoreType.DMA,
            pltpu.SemaphoreType.DMA,
        ],
    )
    def agg(table_hbm, src_hbm, dst4_hbm, norm_hbm, out_hbm,
            src_v, dst_v, norm_v, rows_v, shared, sg0, sg1, sg2, ss0, ss1, ss2):
        c = lax.axis_index("c")
        s = lax.axis_index("s")
        eslot = s * 2 + c if split_edges else s
        ebase = eslot * ept

        def zrow(j, carry):
            for k in range(D2 // 16):
                rows_v[0, j, pl.ds(k * 16, 16)] = jnp.zeros((16,), jnp.float32)
            return carry

        lax.fori_loop(0, BK, zrow, 0)
        nbase = s * 640
        nchunks = jnp.where(s < 15, 8, 5)

        def zchunk(j, carry):
            pltpu.sync_copy(rows_v.at[0], shared.at[pl.ds(nbase + j * BK, BK)])
            return carry

        lax.fori_loop(0, nchunks, zchunk, 0)
        plsc.subcore_barrier()

        tbl = table_hbm if split_edges else table_hbm.at[c]
        sgs = (sg0, sg1, sg2)
        sss = (ss0, ss1, ss2)

        def gather(i, b):
            return pltpu.async_copy(
                tbl.at[src_v.at[pl.ds(i * BK, BK)]], rows_v.at[b], sgs[b])

        def wait_gather(i, b):
            pltpu.make_async_copy(
                tbl.at[src_v.at[pl.ds(i * BK, BK)]], rows_v.at[b], sgs[b]).wait()

        def wait_scatter(i, b):
            pltpu.make_async_copy(
                rows_v.at[b], shared.at[dst_v.at[i]], sss[b]).wait()

        def scale(i, b):
            def ebody(e, carry):
                nsp = plsc.load_gather(
                    norm_v, [jnp.full((16,), i * BK + e, jnp.int32)])
                for k in range(D2 // 16):
                    sl = pl.ds(k * 16, 16)
                    rows_v[b, e, sl] = rows_v[b, e, sl] * nsp
                return carry

            lax.fori_loop(0, BK, ebody, 0, unroll=8)

        def group_body(g, carry):
            gb = pl.multiple_of(ebase + g * (GB * BK), 8)
            pltpu.sync_copy(src_hbm.at[pl.ds(gb, GB * BK)], src_v)
            pltpu.sync_copy(dst4_hbm.at[eslot, g], dst_v)
            pltpu.sync_copy(norm_hbm.at[pl.ds(gb, GB * BK)], norm_v)
            gather(0, 0)
            gather(1, 1)
            gather(2, 2)

            def tri_body(blk, carry2):
                base_i = blk * 3
                for b in (0, 1, 2):
                    i = base_i + b
                    wait_gather(i, b)
                    scale(i, b)
                    pltpu.sync_copy(rows_v.at[b], shared.at[dst_v.at[i]],
                                    add=True)

                    @pl.when(i + 3 < GB)
                    def _():
                        gather(i + 3, b)
                return carry2

            lax.fori_loop(0, GB // 3, tri_body, 0)
            wait_gather(GB - 1, 0)
            scale(GB - 1, 0)
            pltpu.sync_copy(rows_v.at[0], shared.at[dst_v.at[GB - 1]], add=True)
            return carry

        lax.fori_loop(0, ngroups, group_body, 0)
        plsc.subcore_barrier()
        out_c = out_hbm.at[c]

        def wchunk(j, carry):
            sl = pl.ds(nbase + j * BK, BK)
            pltpu.sync_copy(shared.at[sl], out_c.at[sl])
            return carry

        lax.fori_loop(0, nchunks, wchunk, 0)

    return agg


_agg_conv1 = _make_aggregate(True)
_agg_conv2 = _make_aggregate(False)


def _conv_mm_body(agg_ref, x_ref, dinv_ref, W1_ref, b1_ref, W2_ref, out_ref):
    d = dinv_ref[...]
    a = agg_ref[0] + agg_ref[1] + (d * d) * x_ref[...]
    h = jnp.maximum(
        jnp.dot(a, W1_ref[...], preferred_element_type=jnp.float32) + b1_ref[...],
        0.0,
    )
    g = jnp.dot(h, W2_ref[...], preferred_element_type=jnp.float32)
    out_ref[0] = g[:, :128]
    out_ref[1] = g[:, 128:]


def _head_body(agg_ref, g_ref, dinv_ref, b2_ref, Wl1_ref, bl1_ref,
               Wl2_ref, bl2_ref, Wl3_ref, bl3_ref, out_ref):
    d = dinv_ref[...]
    d2 = d * d
    h0 = jnp.maximum(agg_ref[0] + d2 * g_ref[0] + b2_ref[:, :128], 0.0)
    h1 = jnp.maximum(agg_ref[1] + d2 * g_ref[1] + b2_ref[:, 128:], 0.0)
    h = (jnp.dot(h0, Wl1_ref[:128], preferred_element_type=jnp.float32)
         + jnp.dot(h1, Wl1_ref[128:], preferred_element_type=jnp.float32)
         + bl1_ref[...])
    h = jnp.maximum(h, 0.0)
    h = jnp.maximum(
        jnp.dot(h, Wl2_ref[...], preferred_element_type=jnp.float32) + bl2_ref[...], 0.0)
    logits = jnp.dot(h, Wl3_ref[...], preferred_element_type=jnp.float32) + bl3_ref[...]
    m = jnp.max(logits, axis=1, keepdims=True)
    e = jnp.exp(logits - m)
    out_ref[...] = e / jnp.sum(e, axis=1, keepdims=True)


def _row_spec(cols):
    return pl.BlockSpec((RB, cols), lambda i: (i, 0))


def _full_spec(shape):
    return pl.BlockSpec(shape, lambda i: tuple(0 for _ in shape))


def _split_spec(cols):
    return pl.BlockSpec((2, RB, cols), lambda i: (0, i, 0))


def _conv_mm(agg2, x, dinv2, W1, b1, W2):
    return pl.pallas_call(
        _conv_mm_body,
        grid=(N // RB,),
        in_specs=[
            _split_spec(128), _row_spec(128), _row_spec(1),
            _full_spec((128, 512)), _full_spec((1, 512)), _full_spec((512, 256)),
        ],
        out_specs=_split_spec(128),
        out_shape=jax.ShapeDtypeStruct((2, N, 128), jnp.float32),
    )(agg2, x, dinv2, W1, b1.reshape(1, 512), W2)


def _head(agg, g, dinv2, b2, Wl1, bl1, Wl2, bl2, Wl3, bl3):
    return pl.pallas_call(
        _head_body,
        grid=(N // RB,),
        in_specs=[
            _split_spec(128), _split_spec(128), _row_spec(1),
            _full_spec((1, 256)),
            _full_spec((256, 128)), _full_spec((1, 128)),
            _full_spec((128, 64)), _full_spec((1, 64)),
            _full_spec((64, 40)), _full_spec((1, 40)),
        ],
        out_specs=_row_spec(40),
        out_shape=jax.ShapeDtypeStruct((N, 40), jnp.float32),
    )(agg, g, dinv2, b2.reshape(1, 256), Wl1, bl1.reshape(1, 128),
      Wl2, bl2.reshape(1, 64), Wl3, bl3.reshape(1, 40))


def kernel(x, edge_index, edge_weight, W1, b1, W2, b2, Wl1, bl1, Wl2, bl2, Wl3, bl3):
    src = edge_index[0]
    dst = edge_index[1]
    parts = _deg_partials(dst, edge_weight)
    norm, dinv = _edge_norm(parts, src, dst, edge_weight)
    dinv = dinv.reshape(N, 1)

    agg1 = _agg_conv1(x, src, dst.reshape(32, E // (32 * GB * BK), GB, BK), norm)
    g2 = _conv_mm(agg1, x, dinv, W1, b1, W2)
    agg2 = _agg_conv2(g2, src, dst.reshape(16, E // (16 * GB * BK), GB, BK), norm)
    return _head(agg2, g2, dinv, b2, Wl1, bl1, Wl2, bl2, Wl3, bl3)
